# baseline (speedup 1.0000x reference)
import functools

import jax
import jax.numpy as jnp
from jax import lax
from jax.experimental import pallas as pl
from jax.experimental.pallas import tpu as pltpu
from jax.experimental.pallas import tpu_sc as plsc

_CH = 128


def _sc_gather(B, N, D, NC, NS):
    NW = NC * NS
    nw_ch = B // (NW * _CH)
    nc_ch = N // (NW * _CH)
    mesh = plsc.VectorSubcoreMesh(core_axis_name="c", subcore_axis_name="s")

    @functools.partial(
        pl.kernel,
        out_type=(
            jax.ShapeDtypeStruct((B, D), jnp.float32),
            jax.ShapeDtypeStruct((N, D), jnp.float32),
        ),
        mesh=mesh,
        compiler_params=pltpu.CompilerParams(use_tc_tiling_on_sc=False),
        scratch_types=[
            pltpu.VMEM((nw_ch, _CH), jnp.int32),
            pltpu.VMEM((nc_ch, _CH), jnp.int32),
            pltpu.VMEM((2, _CH, D), jnp.float32),
            pltpu.SemaphoreType.DMA,
        ],
    )
    def body(widx_hbm, cidx_hbm, wtab, ctab, w_out, c_out, widx_v, cidx_v, rows_v, sem):
        wid = lax.axis_index("s") * NC + lax.axis_index("c")
        pltpu.sync_copy(widx_hbm.at[wid], widx_v)
        pltpu.sync_copy(cidx_hbm.at[wid], cidx_v)

        w_base = wid * (nw_ch * _CH)
        for j in range(nw_ch):
            pltpu.async_copy(wtab.at[widx_v.at[j]], rows_v.at[0], sem).wait()
            pltpu.sync_copy(rows_v.at[0], w_out.at[pl.ds(w_base + j * _CH, _CH)])

        c_base = wid * (nc_ch * _CH)
        def chunk(j, carry):
            pltpu.async_copy(ctab.at[cidx_v.at[j]], rows_v.at[0], sem).wait()
            pltpu.sync_copy(rows_v.at[0], c_out.at[pl.ds(c_base + j * _CH, _CH)])
            return carry
        lax.fori_loop(0, nc_ch, chunk, 0)

    return body


def kernel(words, contexts, w_embedding, c_embedding):
    (B,) = words.shape
    _, CTX = contexts.shape
    _, D = w_embedding.shape
    N = B * CTX
    info = plsc.get_sparse_core_info()
    NC, NS = info.num_cores, info.num_subcores
    NW = NC * NS

    w_idx = words.reshape(NW, B // (NW * _CH), _CH)
    c_idx = contexts.reshape(NW, N // (NW * _CH), _CH)
    w_out, c_out = _sc_gather(B, N, D, NC, NS)(w_idx, c_idx, w_embedding, c_embedding)
    return w_out, c_out.reshape(B, CTX, D)


# speedup vs baseline: 1.0456x; 1.0456x over previous
import functools

import jax
import jax.numpy as jnp
from jax import lax
from jax.experimental import pallas as pl
from jax.experimental.pallas import tpu as pltpu
from jax.experimental.pallas import tpu_sc as plsc

_CH = 128
_SUB = 4
_SUP = _CH * _SUB


def _sc_gather(B, N, D, NC, NS):
    NW = NC * NS
    nw_ch = B // (NW * _CH)
    nc_ch = N // (NW * _CH)
    nc_sup = nc_ch // _SUB
    assert nw_ch == _SUB and nc_ch % _SUB == 0
    mesh = plsc.VectorSubcoreMesh(core_axis_name="c", subcore_axis_name="s")

    @functools.partial(
        pl.kernel,
        out_type=(
            jax.ShapeDtypeStruct((B, D), jnp.float32),
            jax.ShapeDtypeStruct((N, D), jnp.float32),
        ),
        mesh=mesh,
        compiler_params=pltpu.CompilerParams(use_tc_tiling_on_sc=False),
        scratch_types=[
            pltpu.VMEM((nw_ch, _CH), jnp.int32),
            pltpu.VMEM((nc_ch, _CH), jnp.int32),
            pltpu.VMEM((2, _SUP, D), jnp.float32),
            pltpu.SemaphoreType.DMA,
            pltpu.SemaphoreType.DMA,
        ],
    )
    def body(widx_hbm, cidx_hbm, wtab, ctab, w_out, c_out, widx_v, cidx_v, rows_v, sem0, sem1):
        wid = lax.axis_index("s") * NC + lax.axis_index("c")
        pltpu.sync_copy(widx_hbm.at[wid], widx_v)
        pltpu.sync_copy(cidx_hbm.at[wid], cidx_v)

        w_base = wid * _SUP
        c_base = wid * (nc_sup * _SUP)
        sems = (sem0, sem1)

        def fire(tab, idx_v, sup, buf):
            for k in range(_SUB):
                pltpu.async_copy(
                    tab.at[idx_v.at[sup * _SUB + k]],
                    rows_v.at[buf, pl.ds(k * _CH, _CH)],
                    sems[buf],
                )

        def drain_write(tab, idx_v, sup, buf, out, out_base):
            for k in range(_SUB):
                pltpu.make_async_copy(
                    tab.at[idx_v.at[sup * _SUB + k]],
                    rows_v.at[buf, pl.ds(k * _CH, _CH)],
                    sems[buf],
                ).wait()
            pltpu.sync_copy(rows_v.at[buf], out.at[pl.ds(out_base, _SUP)])

        fire(wtab, widx_v, 0, 0)
        fire(ctab, cidx_v, 0, 1)
        drain_write(wtab, widx_v, 0, 0, w_out, w_base)

        def step(p, carry):
            fire(ctab, cidx_v, 2 * p + 1, 0)
            drain_write(ctab, cidx_v, 2 * p, 1, c_out, c_base + (2 * p) * _SUP)
            fire(ctab, cidx_v, 2 * p + 2, 1)
            drain_write(ctab, cidx_v, 2 * p + 1, 0, c_out, c_base + (2 * p + 1) * _SUP)
            return carry

        assert nc_sup % 2 == 0 and nc_sup >= 4
        lax.fori_loop(0, nc_sup // 2 - 1, step, 0)

        last = nc_sup - 1
        fire(ctab, cidx_v, last, 0)
        drain_write(ctab, cidx_v, last - 1, 1, c_out, c_base + (last - 1) * _SUP)
        drain_write(ctab, cidx_v, last, 0, c_out, c_base + last * _SUP)

    return body


def kernel(words, contexts, w_embedding, c_embedding):
    (B,) = words.shape
    _, CTX = contexts.shape
    _, D = w_embedding.shape
    N = B * CTX
    info = plsc.get_sparse_core_info()
    NC, NS = info.num_cores, info.num_subcores
    NW = NC * NS

    w_idx = words.reshape(NW, B // (NW * _CH), _CH)
    c_idx = contexts.reshape(NW, N // (NW * _CH), _CH)
    w_out, c_out = _sc_gather(B, N, D, NC, NS)(w_idx, c_idx, w_embedding, c_embedding)
    return w_out, c_out.reshape(B, CTX, D)


# speedup vs baseline: 1.0686x; 1.0220x over previous
import functools

import jax
import jax.numpy as jnp
from jax import lax
from jax.experimental import pallas as pl
from jax.experimental.pallas import tpu as pltpu
from jax.experimental.pallas import tpu_sc as plsc

_CH = 128
_SUB = 4
_SUP = _CH * _SUB


def _sc_gather(B, N, D, NC, NS):
    NW = NC * NS
    nw_ch = B // (NW * _CH)
    nc_ch = N // (NW * _CH)
    nc_sup = nc_ch // _SUB
    assert nw_ch == _SUB and nc_ch % _SUB == 0
    mesh = plsc.VectorSubcoreMesh(core_axis_name="c", subcore_axis_name="s")

    @functools.partial(
        pl.kernel,
        out_type=(
            jax.ShapeDtypeStruct((B, D), jnp.float32),
            jax.ShapeDtypeStruct((N, D), jnp.float32),
        ),
        mesh=mesh,
        compiler_params=pltpu.CompilerParams(use_tc_tiling_on_sc=False),
        scratch_types=[
            pltpu.VMEM((nw_ch, _CH), jnp.int32),
            pltpu.VMEM((nc_ch, _CH), jnp.int32),
            pltpu.VMEM((2, _SUP, D), jnp.float32),
            pltpu.SemaphoreType.DMA,
            pltpu.SemaphoreType.DMA,
        ],
    )
    def body(widx_hbm, cidx_hbm, wtab, ctab, w_out, c_out, widx_v, cidx_v, rows_v, sem0, sem1):
        wid = lax.axis_index("s") * NC + lax.axis_index("c")
        pltpu.sync_copy(widx_hbm.at[wid], widx_v)
        pltpu.sync_copy(cidx_hbm.at[wid], cidx_v)

        w_base = wid * _SUP
        c_base = wid * (nc_sup * _SUP)
        sems = (sem0, sem1)

        def fire(tab, idx_v, sup, buf):
            for k in range(_SUB):
                pltpu.async_copy(
                    tab.at[idx_v.at[sup * _SUB + k]],
                    rows_v.at[buf, pl.ds(k * _CH, _CH)],
                    sems[buf],
                )

        def drain_write(tab, idx_v, sup, buf, out, out_base):
            for k in range(_SUB):
                pltpu.make_async_copy(
                    tab.at[idx_v.at[sup * _SUB + k]],
                    rows_v.at[buf, pl.ds(k * _CH, _CH)],
                    sems[buf],
                ).wait()
            pltpu.sync_copy(rows_v.at[buf], out.at[pl.ds(out_base, _SUP)])

        fire(wtab, widx_v, 0, 0)
        fire(ctab, cidx_v, 0, 1)
        drain_write(wtab, widx_v, 0, 0, w_out, w_base)

        def step(p, carry):
            fire(ctab, cidx_v, 2 * p + 1, 0)
            drain_write(ctab, cidx_v, 2 * p, 1, c_out, c_base + (2 * p) * _SUP)
            fire(ctab, cidx_v, 2 * p + 2, 1)
            drain_write(ctab, cidx_v, 2 * p + 1, 0, c_out, c_base + (2 * p + 1) * _SUP)
            return carry

        assert nc_sup % 2 == 0 and nc_sup >= 4
        lax.fori_loop(0, nc_sup // 2 - 1, step, 0)

        last = nc_sup - 1
        fire(ctab, cidx_v, last, 0)
        drain_write(ctab, cidx_v, last - 1, 1, c_out, c_base + (last - 1) * _SUP)
        drain_write(ctab, cidx_v, last, 0, c_out, c_base + last * _SUP)

    return body


def kernel(words, contexts, w_embedding, c_embedding):
    (B,) = words.shape
    _, CTX = contexts.shape
    _, D = w_embedding.shape
    N = B * CTX
    info = plsc.get_sparse_core_info()
    NC, NS = info.num_cores, info.num_subcores
    NW = NC * NS

    w_idx = words.reshape(NW, B // (NW * _CH), _CH)
    c_idx = contexts.T.reshape(NW, N // (NW * _CH), _CH)
    w_out, c_out = _sc_gather(B, N, D, NC, NS)(w_idx, c_idx, w_embedding, c_embedding)
    return w_out, c_out.reshape(CTX, B, D).transpose(1, 0, 2)


# speedup vs baseline: 1.6914x; 1.5829x over previous
import functools

import jax
import jax.numpy as jnp
from jax import lax
from jax.experimental import pallas as pl
from jax.experimental.pallas import tpu as pltpu
from jax.experimental.pallas import tpu_sc as plsc

_CCH = 256
_WCH = 64
_L = 16


def _sc_gather_c(N, D, NC, NS):
    NW = NC * NS
    bc = N // NW
    n_ch = bc // _CCH
    assert bc % _CCH == 0 and n_ch % 2 == 0 and n_ch >= 4
    mesh = plsc.VectorSubcoreMesh(core_axis_name="c", subcore_axis_name="s")

    @functools.partial(
        pl.kernel,
        out_type=jax.ShapeDtypeStruct((N, D), jnp.float32),
        mesh=mesh,
        compiler_params=pltpu.CompilerParams(
            use_tc_tiling_on_sc=True, needs_layout_passes=False
        ),
        scratch_types=[
            pltpu.VMEM((bc,), jnp.int32),
            pltpu.VMEM((2, _CCH, D), jnp.float32),
            pltpu.SemaphoreType.DMA,
            pltpu.SemaphoreType.DMA,
        ],
    )
    def body(cidx_hbm, ctab, c_out, cidx_v, rows_v, sem0, sem1):
        wid = lax.axis_index("s") * NC + lax.axis_index("c")
        pltpu.sync_copy(cidx_hbm.at[wid], cidx_v)
        c_base = wid * bc
        sems = (sem0, sem1)

        def issue(chunk, buf):
            def blk(j, carry):
                v16 = cidx_v[pl.ds(chunk * _CCH + j * _L, _L)]
                for k in range(_L):
                    pltpu.async_copy(
                        ctab.at[pl.ds(v16[k], 1)],
                        rows_v.at[buf, pl.ds(j * _L + k, 1)],
                        sems[buf],
                    )
                return carry

            lax.fori_loop(0, _CCH // _L, blk, 0)

        def drain_write(chunk, buf):
            pltpu.make_async_copy(
                c_out.at[pl.ds(0, _CCH)], rows_v.at[buf], sems[buf]
            ).wait()
            pltpu.sync_copy(
                rows_v.at[buf], c_out.at[pl.ds(c_base + chunk * _CCH, _CCH)]
            )

        issue(0, 0)
        issue(1, 1)

        def step(p, carry):
            drain_write(2 * p, 0)
            issue(2 * p + 2, 0)
            drain_write(2 * p + 1, 1)
            issue(2 * p + 3, 1)
            return carry

        lax.fori_loop(0, n_ch // 2 - 1, step, 0)
        drain_write(n_ch - 2, 0)
        drain_write(n_ch - 1, 1)

    return body


def _sc_gather_w(B, D, V, NC, NS):
    NW = NC * NS
    bw = B // NW
    n_ch = bw // _WCH
    assert bw % _WCH == 0 and V % 8 == 0
    mesh = plsc.VectorSubcoreMesh(core_axis_name="c", subcore_axis_name="s")

    @functools.partial(
        pl.kernel,
        out_type=jax.ShapeDtypeStruct((B, D), jnp.float32),
        mesh=mesh,
        compiler_params=pltpu.CompilerParams(
            use_tc_tiling_on_sc=True, needs_layout_passes=False
        ),
        scratch_types=[
            pltpu.VMEM((bw,), jnp.int32),
            pltpu.VMEM((bw, D), jnp.float32),
            pltpu.SemaphoreType.DMA,
        ],
    )
    def body(widx_hbm, wtab, w_out, widx_v, rows_v, sem):
        wid = lax.axis_index("s") * NC + lax.axis_index("c")
        pltpu.sync_copy(widx_hbm.at[wid], widx_v)

        def issue(j, carry):
            v16 = widx_v[pl.ds(j * _L, _L)]
            for k in range(_L):
                pltpu.async_copy(
                    wtab.at[pl.ds(v16[k], 1)],
                    rows_v.at[pl.ds(j * _L + k, 1)],
                    sem,
                )
            return carry

        lax.fori_loop(0, bw // _L, issue, 0)
        pltpu.make_async_copy(
            w_out.at[pl.ds(0, bw)], rows_v, sem
        ).wait()
        pltpu.sync_copy(rows_v, w_out.at[pl.ds(wid * bw, bw)])

    return body


def kernel(words, contexts, w_embedding, c_embedding):
    (B,) = words.shape
    _, CTX = contexts.shape
    V, D = w_embedding.shape
    N = B * CTX
    info = plsc.get_sparse_core_info()
    NC, NS = info.num_cores, info.num_subcores
    NW = NC * NS

    w_idx = words.reshape(NW, B // NW)
    c_idx = contexts.T.reshape(NW, N // NW)
    w_out = _sc_gather_w(B, D, V, NC, NS)(w_idx, w_embedding)
    c_out = _sc_gather_c(N, D, NC, NS)(c_idx, c_embedding)
    return w_out, c_out.reshape(CTX, B, D).transpose(1, 0, 2)
